# baseline (speedup 1.0000x reference)
import functools

import jax
import jax.numpy as jnp
from jax import lax
from jax.experimental import pallas as pl
from jax.experimental.pallas import tpu as pltpu
from jax.experimental.pallas import tpu_sc as plsc

_N_EMB = 8192
_D = 256
_B, _C, _H, _Wd = 8, 64, 4, 2048
_N = _B * _Wd
_TI = 256
_G = _N // _TI

_NC, _NS = 2, 16
_NW = _NC * _NS
_ROWS_PER_W = _N // _NW
_CH = 256


def _dist_body(zp_ref, wt_ref, w2_ref, idx_ref, psum_ref):
    zp = zp_ref[...]
    zb = zp.astype(jnp.bfloat16).astype(jnp.float32)
    zw = jnp.dot(zb, wt_ref[...], preferred_element_type=jnp.float32,
                 precision=lax.Precision.HIGHEST)
    z2 = jnp.sum(zp * zp, axis=1, keepdims=True)
    d = (z2 + w2_ref[0, :][None, :]) - 2.0 * zw
    m = jnp.min(d, axis=1, keepdims=True)
    ids = lax.broadcasted_iota(jnp.int32, (_TI, _N_EMB), 1)
    idx = jnp.min(jnp.where(d == m, ids, jnp.int32(_N_EMB)), axis=1)
    idx_ref[0, 0, :] = idx
    psum_ref[0, 0, :] = jnp.broadcast_to(jnp.sum(m), (128,))


def _distance_argmin(zp_flat, wt, w2row):
    return pl.pallas_call(
        _dist_body,
        grid=(_G,),
        in_specs=[
            pl.BlockSpec((_TI, _D), lambda i: (i, 0)),
            pl.BlockSpec((_D, _N_EMB), lambda i: (0, 0)),
            pl.BlockSpec((1, _N_EMB), lambda i: (0, 0)),
        ],
        out_specs=[
            pl.BlockSpec((1, 1, _TI), lambda i: (i, 0, 0)),
            pl.BlockSpec((1, 1, 128), lambda i: (i, 0, 0)),
        ],
        out_shape=[
            jax.ShapeDtypeStruct((_G, 1, _TI), jnp.int32),
            jax.ShapeDtypeStruct((_G, 1, 128), jnp.float32),
        ],
    )(zp_flat, wt, w2row)


def _sc_gather(table, idx_flat):
    mesh = plsc.VectorSubcoreMesh(core_axis_name="c", subcore_axis_name="s")

    @functools.partial(
        pl.kernel,
        mesh=mesh,
        out_type=jax.ShapeDtypeStruct((_N, _D), jnp.float32),
        scratch_types=[
            pltpu.VMEM((_CH,), jnp.int32),
            pltpu.VMEM((_CH, _D), jnp.float32),
            pltpu.SemaphoreType.DMA,
        ],
    )
    def k(table_hbm, idx_hbm, out_hbm, idx_v, rows_v, sem):
        wid = lax.axis_index("s") * _NC + lax.axis_index("c")
        base = wid * _ROWS_PER_W
        for chunk in range(_ROWS_PER_W // _CH):
            off = base + chunk * _CH
            pltpu.sync_copy(idx_hbm.at[pl.ds(off, _CH)], idx_v)
            pltpu.async_copy(table_hbm.at[idx_v], rows_v, sem).wait()
            pltpu.sync_copy(rows_v, out_hbm.at[pl.ds(off, _CH)])

    return k(table, idx_flat)


def kernel(z, W):
    zp_flat = jnp.transpose(z, (0, 3, 1, 2)).reshape(_N, _D)
    wt = W.T
    w2row = jnp.sum(W ** 2, axis=1)[None, :]
    idx3, psum = _distance_argmin(zp_flat, wt, w2row)
    idx_flat = idx3.reshape(_N)
    zq_flat = _sc_gather(W, idx_flat)
    z_q = jnp.transpose(zq_flat.reshape(_B, _Wd, _C, _H), (0, 2, 3, 1))
    loss = jnp.sum(psum[:, 0, 0]) * jnp.float32(1.25 / (_N * _D))
    return z_q, loss, idx_flat.reshape(_B, _Wd)


# speedup vs baseline: 2.3252x; 2.3252x over previous
import functools

import jax
import jax.numpy as jnp
from jax import lax
from jax.experimental import pallas as pl
from jax.experimental.pallas import tpu as pltpu
from jax.experimental.pallas import tpu_sc as plsc

_N_EMB = 8192
_D = 256
_B, _C, _H, _Wd = 8, 64, 4, 2048
_N = _B * _Wd
_TI = 256
_G = _N // _TI

_NC, _NS = 2, 16
_NW = _NC * _NS
_ROWS_PER_W = _N // _NW
_CH = 256


def _dist_body(zp_ref, wt_ref, w2_ref, idx_ref, psum_ref):
    zp = zp_ref[...]
    zb = zp.astype(jnp.bfloat16).astype(jnp.float32)
    zw = jnp.dot(zb, wt_ref[...], preferred_element_type=jnp.float32,
                 precision=lax.Precision.DEFAULT)
    z2 = jnp.sum(zp * zp, axis=1, keepdims=True)
    d = (z2 + w2_ref[0, :][None, :]) - 2.0 * zw
    m = jnp.min(d, axis=1, keepdims=True)
    ids = lax.broadcasted_iota(jnp.int32, (_TI, _N_EMB), 1)
    idx = jnp.min(jnp.where(d == m, ids, jnp.int32(_N_EMB)), axis=1)
    idx_ref[0, 0, :] = idx
    psum_ref[0, 0, :] = jnp.broadcast_to(jnp.sum(m), (128,))


def _distance_argmin(zp_flat, wt, w2row):
    return pl.pallas_call(
        _dist_body,
        grid=(_G,),
        in_specs=[
            pl.BlockSpec((_TI, _D), lambda i: (i, 0)),
            pl.BlockSpec((_D, _N_EMB), lambda i: (0, 0)),
            pl.BlockSpec((1, _N_EMB), lambda i: (0, 0)),
        ],
        out_specs=[
            pl.BlockSpec((1, 1, _TI), lambda i: (i, 0, 0)),
            pl.BlockSpec((1, 1, 128), lambda i: (i, 0, 0)),
        ],
        out_shape=[
            jax.ShapeDtypeStruct((_G, 1, _TI), jnp.int32),
            jax.ShapeDtypeStruct((_G, 1, 128), jnp.float32),
        ],
    )(zp_flat, wt, w2row)


def _sc_gather(table, idx_flat):
    mesh = plsc.VectorSubcoreMesh(core_axis_name="c", subcore_axis_name="s")

    @functools.partial(
        pl.kernel,
        mesh=mesh,
        out_type=jax.ShapeDtypeStruct((_N, _D), jnp.float32),
        scratch_types=[
            pltpu.VMEM((_CH,), jnp.int32),
            pltpu.VMEM((_CH, _D), jnp.float32),
            pltpu.SemaphoreType.DMA,
        ],
    )
    def k(table_hbm, idx_hbm, out_hbm, idx_v, rows_v, sem):
        wid = lax.axis_index("s") * _NC + lax.axis_index("c")
        base = wid * _ROWS_PER_W
        for chunk in range(_ROWS_PER_W // _CH):
            off = base + chunk * _CH
            pltpu.sync_copy(idx_hbm.at[pl.ds(off, _CH)], idx_v)
            pltpu.async_copy(table_hbm.at[idx_v], rows_v, sem).wait()
            pltpu.sync_copy(rows_v, out_hbm.at[pl.ds(off, _CH)])

    return k(table, idx_flat)


def kernel(z, W):
    zp_flat = jnp.transpose(z, (0, 3, 1, 2)).reshape(_N, _D)
    wt = W.T
    w2row = jnp.sum(W ** 2, axis=1)[None, :]
    idx3, psum = _distance_argmin(zp_flat, wt, w2row)
    idx_flat = idx3.reshape(_N)
    zq_flat = _sc_gather(W, idx_flat)
    z_q = jnp.transpose(zq_flat.reshape(_B, _Wd, _C, _H), (0, 2, 3, 1))
    loss = jnp.sum(psum[:, 0, 0]) * jnp.float32(1.25 / (_N * _D))
    return z_q, loss, idx_flat.reshape(_B, _Wd)


# speedup vs baseline: 2.3887x; 1.0273x over previous
import functools

import jax
import jax.numpy as jnp
from jax import lax
from jax.experimental import pallas as pl
from jax.experimental.pallas import tpu as pltpu
from jax.experimental.pallas import tpu_sc as plsc

_N_EMB = 8192
_D = 256
_B, _C, _H, _Wd = 8, 64, 4, 2048
_N = _B * _Wd
_TI = 512
_G = _N // _TI

_NC, _NS = 2, 16
_NW = _NC * _NS
_ROWS_PER_W = _N // _NW
_CH = 256


def _dist_body(zp_ref, wt_ref, w2_ref, idx_ref, psum_ref):
    zp = zp_ref[...]
    zb = zp.astype(jnp.bfloat16).astype(jnp.float32)
    zw = jnp.dot(zb, wt_ref[...], preferred_element_type=jnp.float32,
                 precision=lax.Precision.DEFAULT)
    z2 = jnp.sum(zp * zp, axis=1, keepdims=True)
    d = (z2 + w2_ref[0, :][None, :]) - 2.0 * zw
    m = jnp.min(d, axis=1, keepdims=True)
    ids = lax.broadcasted_iota(jnp.int32, (_TI, _N_EMB), 1)
    idx = jnp.min(jnp.where(d == m, ids, jnp.int32(_N_EMB)), axis=1)
    idx_ref[0, 0, :] = idx
    psum_ref[0, 0, :] = jnp.broadcast_to(jnp.sum(m), (128,))


def _distance_argmin(zp_flat, wt, w2row):
    return pl.pallas_call(
        _dist_body,
        grid=(_G,),
        in_specs=[
            pl.BlockSpec((_TI, _D), lambda i: (i, 0)),
            pl.BlockSpec((_D, _N_EMB), lambda i: (0, 0)),
            pl.BlockSpec((1, _N_EMB), lambda i: (0, 0)),
        ],
        out_specs=[
            pl.BlockSpec((1, 1, _TI), lambda i: (i, 0, 0)),
            pl.BlockSpec((1, 1, 128), lambda i: (i, 0, 0)),
        ],
        out_shape=[
            jax.ShapeDtypeStruct((_G, 1, _TI), jnp.int32),
            jax.ShapeDtypeStruct((_G, 1, 128), jnp.float32),
        ],
    )(zp_flat, wt, w2row)


def _sc_gather(table, idx_flat):
    mesh = plsc.VectorSubcoreMesh(core_axis_name="c", subcore_axis_name="s")

    @functools.partial(
        pl.kernel,
        mesh=mesh,
        out_type=jax.ShapeDtypeStruct((_N, _D), jnp.float32),
        scratch_types=[
            pltpu.VMEM((_CH,), jnp.int32),
            pltpu.VMEM((_CH, _D), jnp.float32),
            pltpu.SemaphoreType.DMA,
        ],
    )
    def k(table_hbm, idx_hbm, out_hbm, idx_v, rows_v, sem):
        wid = lax.axis_index("s") * _NC + lax.axis_index("c")
        base = wid * _ROWS_PER_W
        for chunk in range(_ROWS_PER_W // _CH):
            off = base + chunk * _CH
            pltpu.sync_copy(idx_hbm.at[pl.ds(off, _CH)], idx_v)
            pltpu.async_copy(table_hbm.at[idx_v], rows_v, sem).wait()
            pltpu.sync_copy(rows_v, out_hbm.at[pl.ds(off, _CH)])

    return k(table, idx_flat)


def kernel(z, W):
    zp_flat = jnp.transpose(z, (0, 3, 1, 2)).reshape(_N, _D)
    wt = W.T
    w2row = jnp.sum(W ** 2, axis=1)[None, :]
    idx3, psum = _distance_argmin(zp_flat, wt, w2row)
    idx_flat = idx3.reshape(_N)
    zq_flat = _sc_gather(W, idx_flat)
    z_q = jnp.transpose(zq_flat.reshape(_B, _Wd, _C, _H), (0, 2, 3, 1))
    loss = jnp.sum(psum[:, 0, 0]) * jnp.float32(1.25 / (_N * _D))
    return z_q, loss, idx_flat.reshape(_B, _Wd)


# speedup vs baseline: 2.4259x; 1.0156x over previous
import functools

import jax
import jax.numpy as jnp
from jax import lax
from jax.experimental import pallas as pl
from jax.experimental.pallas import tpu as pltpu
from jax.experimental.pallas import tpu_sc as plsc

_N_EMB = 8192
_D = 256
_B, _C, _H, _Wd = 8, 64, 4, 2048
_N = _B * _Wd
_TI = 1024
_G = _N // _TI

_NC, _NS = 2, 16
_NW = _NC * _NS
_ROWS_PER_W = _N // _NW
_CH = 256


def _dist_body(zp_ref, wt_ref, w2_ref, idx_ref, psum_ref):
    zp = zp_ref[...]
    zb = zp.astype(jnp.bfloat16).astype(jnp.float32)
    zw = jnp.dot(zb, wt_ref[...], preferred_element_type=jnp.float32,
                 precision=lax.Precision.DEFAULT)
    z2 = jnp.sum(zp * zp, axis=1, keepdims=True)
    d = (z2 + w2_ref[0, :][None, :]) - 2.0 * zw
    m = jnp.min(d, axis=1, keepdims=True)
    ids = lax.broadcasted_iota(jnp.int32, (_TI, _N_EMB), 1)
    idx = jnp.min(jnp.where(d == m, ids, jnp.int32(_N_EMB)), axis=1)
    idx_ref[0, 0, :] = idx
    psum_ref[0, 0, :] = jnp.broadcast_to(jnp.sum(m), (128,))


def _distance_argmin(zp_flat, wt, w2row):
    return pl.pallas_call(
        _dist_body,
        grid=(_G,),
        in_specs=[
            pl.BlockSpec((_TI, _D), lambda i: (i, 0)),
            pl.BlockSpec((_D, _N_EMB), lambda i: (0, 0)),
            pl.BlockSpec((1, _N_EMB), lambda i: (0, 0)),
        ],
        out_specs=[
            pl.BlockSpec((1, 1, _TI), lambda i: (i, 0, 0)),
            pl.BlockSpec((1, 1, 128), lambda i: (i, 0, 0)),
        ],
        out_shape=[
            jax.ShapeDtypeStruct((_G, 1, _TI), jnp.int32),
            jax.ShapeDtypeStruct((_G, 1, 128), jnp.float32),
        ],
    )(zp_flat, wt, w2row)


def _sc_gather(table, idx_flat):
    mesh = plsc.VectorSubcoreMesh(core_axis_name="c", subcore_axis_name="s")

    @functools.partial(
        pl.kernel,
        mesh=mesh,
        out_type=jax.ShapeDtypeStruct((_N, _D), jnp.float32),
        scratch_types=[
            pltpu.VMEM((_CH,), jnp.int32),
            pltpu.VMEM((_CH, _D), jnp.float32),
            pltpu.SemaphoreType.DMA,
        ],
    )
    def k(table_hbm, idx_hbm, out_hbm, idx_v, rows_v, sem):
        wid = lax.axis_index("s") * _NC + lax.axis_index("c")
        base = wid * _ROWS_PER_W
        for chunk in range(_ROWS_PER_W // _CH):
            off = base + chunk * _CH
            pltpu.sync_copy(idx_hbm.at[pl.ds(off, _CH)], idx_v)
            pltpu.async_copy(table_hbm.at[idx_v], rows_v, sem).wait()
            pltpu.sync_copy(rows_v, out_hbm.at[pl.ds(off, _CH)])

    return k(table, idx_flat)


def kernel(z, W):
    zp_flat = jnp.transpose(z, (0, 3, 1, 2)).reshape(_N, _D)
    wt = W.T
    w2row = jnp.sum(W ** 2, axis=1)[None, :]
    idx3, psum = _distance_argmin(zp_flat, wt, w2row)
    idx_flat = idx3.reshape(_N)
    zq_flat = _sc_gather(W, idx_flat)
    z_q = jnp.transpose(zq_flat.reshape(_B, _Wd, _C, _H), (0, 2, 3, 1))
    loss = jnp.sum(psum[:, 0, 0]) * jnp.float32(1.25 / (_N * _D))
    return z_q, loss, idx_flat.reshape(_B, _Wd)
